# baseline (speedup 1.0000x reference)
import jax
import jax.numpy as jnp
import numpy as np
from jax.experimental import pallas as pl
from jax.experimental.pallas import tpu as pltpu

N = 1048576
C = 21
B = 32768
NSTEP = N // (2 * B)
RATIO = 3

_W20 = np.concatenate([np.ones((1, 20), np.float32),
                       np.zeros((1, 1), np.float32)], axis=1)

TROWS = 1024
TCOLS = 1024

LOG2E = 1.4426950408889634
LN2 = 0.6931471805599453


def _cutoff_step0(tf_ref, o_ref, r_ref):
    o_ref[0, 0] = 0.0
    tb = tf_ref[...]
    neg = (tb == (C - 1)).astype(jnp.int32)
    n_neg = jnp.sum(neg)
    t_hard = RATIO * (N - n_neg)
    r_ref[0] = N

    @pl.when(t_hard < n_neg)
    def _():
        r0 = jax.lax.broadcasted_iota(jnp.int32, (TROWS, TCOLS), 0)
        r1 = jax.lax.broadcasted_iota(jnp.int32, (TROWS, TCOLS), 1)
        flat = r0 * TCOLS + r1

        def body(_, lohi):
            lo, hi = lohi
            mid = (lo + hi) // 2
            le = jnp.sum(jnp.where(flat <= mid, neg, 0))
            big = le >= t_hard + 1
            return (jnp.where(big, lo, mid + 1), jnp.where(big, mid, hi))

        lo, _ = jax.lax.fori_loop(0, (N - 1).bit_length(), body, (0, N - 1))
        r_ref[0] = lo


def _half(x_ref, t_ref, w_ref, o_ref, r_cut, base):
    x = x_ref[...]
    t = jnp.reshape(t_ref[...], (1, B))

    m = x * LOG2E
    l = jnp.log2(jnp.exp2(m) + 1.0)

    pos = t != (C - 1)
    tmask = jnp.where(pos, t, -1)
    ci = jax.lax.broadcasted_iota(jnp.int32, (C, B), 0)
    y = l - jnp.where(ci == tmask, m, 0.0)

    q = jax.lax.dot_general(
        w_ref[...], y,
        (((1,), (0,)), ((), ())),
        preferred_element_type=jnp.float32)

    @pl.when(base + B <= r_cut)
    def _():
        o_ref[0, 0] += jnp.sum(q) * LN2

    @pl.when(base + B > r_cut)
    def _():
        row = base + jax.lax.broadcasted_iota(jnp.int32, (1, B), 1)
        sel = jnp.logical_or(pos, row < r_cut)
        o_ref[0, 0] += jnp.sum(jnp.where(sel, q, 0.0)) * LN2


def _fused_kernel(xa_ref, xb_ref, ta_ref, tb_ref, tf_ref, w_ref, o_ref, r_ref):
    j = pl.program_id(0)

    @pl.when(j == 0)
    def _():
        _cutoff_step0(tf_ref, o_ref, r_ref)

    r_cut = r_ref[0]
    _half(xa_ref, ta_ref, w_ref, o_ref, r_cut, (2 * j) * B)
    _half(xb_ref, tb_ref, w_ref, o_ref, r_cut, (2 * j + 1) * B)


def kernel(inputs, targets):
    x_t = jnp.transpose(inputs)
    t2 = jnp.reshape(targets, (TROWS, TCOLS))

    out = pl.pallas_call(
        _fused_kernel,
        grid=(NSTEP,),
        in_specs=[
            pl.BlockSpec((C, B), lambda j: (0, 2 * j)),
            pl.BlockSpec((C, B), lambda j: (0, 2 * j + 1)),
            pl.BlockSpec((B,), lambda j: (2 * j,)),
            pl.BlockSpec((B,), lambda j: (2 * j + 1,)),
            pl.BlockSpec((TROWS, TCOLS), lambda j: (0, 0)),
            pl.BlockSpec((1, C), lambda j: (0, 0)),
        ],
        out_specs=pl.BlockSpec((1, 1), lambda j: (0, 0),
                               memory_space=pltpu.SMEM),
        out_shape=jax.ShapeDtypeStruct((1, 1), jnp.float32),
        scratch_shapes=[pltpu.SMEM((1,), jnp.int32)],
    )(x_t, x_t, targets, targets, t2, jnp.asarray(_W20))

    return out[0, 0]
